# Initial kernel scaffold; baseline (speedup 1.0000x reference)
import jax
import jax.numpy as jnp
from jax.experimental import pallas as pl


def kernel(X, W):
    raise NotImplementedError("write your pallas kernel here")


# speedup vs baseline: 1.0740x; 1.0740x over previous
import functools

import jax
import jax.numpy as jnp
from jax import lax
from jax.experimental import pallas as pl
from jax.experimental.pallas import tpu as pltpu
from jax.experimental.pallas import tpu_sc as plsc


@functools.cache
def _make_gather(B, D, CH):
    info = plsc.get_sparse_core_info()
    nc, ns = info.num_cores, info.num_subcores
    NW = nc * ns
    b_per_w = B // NW
    n_ch = b_per_w // CH
    mesh = plsc.VectorSubcoreMesh(core_axis_name="c", subcore_axis_name="s")

    @functools.partial(
        pl.kernel,
        mesh=mesh,
        out_type=jax.ShapeDtypeStruct((B, D), jnp.float32),
        scratch_types=[
            pltpu.VMEM((CH,), jnp.int32),
            pltpu.VMEM((CH, D), jnp.float32),
            pltpu.SemaphoreType.DMA,
        ],
        compiler_params=pltpu.CompilerParams(use_tc_tiling_on_sc=False),
    )
    def k(idx_hbm, table_hbm, out_hbm, idx_v, rows_v, sem):
        wid = lax.axis_index("s") * nc + lax.axis_index("c")
        base = wid * b_per_w

        def body(j, carry):
            off = base + j * CH
            pltpu.sync_copy(idx_hbm.at[pl.ds(off, CH)], idx_v)
            pltpu.async_copy(table_hbm.at[idx_v], rows_v, sem).wait()
            pltpu.sync_copy(rows_v, out_hbm.at[pl.ds(off, CH)])
            return carry

        lax.fori_loop(0, n_ch, body, 0)

    return k


def kernel(X, W):
    B, H = X.shape
    D = W.shape[1]
    idx = X.reshape(B * H).astype(jnp.int32)
    out = _make_gather(B * H, D, 512)(idx, W)
    return out.reshape(B, H, D)


# speedup vs baseline: 2.0883x; 1.9444x over previous
import functools

import jax
import jax.numpy as jnp
from jax import lax
from jax.experimental import pallas as pl
from jax.experimental.pallas import tpu as pltpu
from jax.experimental.pallas import tpu_sc as plsc


@functools.cache
def _make_embed(T, B, D, V):
    info = plsc.get_sparse_core_info()
    nc, ns = info.num_cores, info.num_subcores
    NW = nc * ns
    CH = B // NW
    NCB = CH // 128
    NR = D // 8
    SKEW = CH + 9
    mesh = plsc.VectorSubcoreMesh(core_axis_name="c", subcore_axis_name="s")

    @functools.partial(
        pl.kernel,
        mesh=mesh,
        out_type=jax.ShapeDtypeStruct((T, NR, B // 128, 8, 128), jnp.float32),
        scratch_types=[
            pltpu.VMEM((CH,), jnp.int32),
            pltpu.VMEM((CH, D), jnp.float32),
            pltpu.VMEM((D, SKEW), jnp.float32),
            pltpu.SemaphoreType.DMA,
            pltpu.SemaphoreType.DMA,
        ],
        compiler_params=pltpu.CompilerParams(
            use_tc_tiling_on_sc=False, needs_layout_passes=False
        ),
    )
    def k(idx_hbm, table_hbm, out_hbm, idx_v, rows_v, tr_v, gsem, osem):
        wid = lax.axis_index("s") * nc + lax.axis_index("c")
        base_b = wid * CH

        def per_t(t, carry):
            off = t * B + base_b
            pltpu.sync_copy(idx_hbm.at[pl.ds(off, CH)], idx_v)
            pltpu.async_copy(table_hbm.at[idx_v], rows_v, gsem).wait()

            def tbody(i, c2):
                for u in range(8):
                    b = i * 8 + u
                    b_idx = jnp.full((16,), b, jnp.int32)
                    for j in range(D // 16):
                        x = rows_v[b, pl.ds(j * 16, 16)]
                        d_idx = lax.iota(jnp.int32, 16) + (j * 16)
                        plsc.store_scatter(tr_v, [d_idx, b_idx], x)
                return c2

            lax.fori_loop(0, CH // 8, tbody, 0)

            copies = []
            for r in range(NR):
                for c in range(NCB):
                    src = tr_v.at[pl.ds(r * 8, 8), pl.ds(c * 128, 128)]
                    dst = out_hbm.at[t, r, wid * NCB + c]
                    copies.append(pltpu.async_copy(src, dst, osem))
            for cp in copies:
                cp.wait()
            return carry

        lax.fori_loop(0, T, per_t, 0)

    return k


def kernel(X, W):
    B, T = X.shape
    V, D = W.shape
    idx = X.transpose(1, 0).reshape(T * B).astype(jnp.int32)
    out5 = _make_embed(T, B, D, V)(idx, W)
    return out5.transpose(2, 4, 0, 1, 3).reshape(B, T, D)


# speedup vs baseline: 2.2640x; 1.0841x over previous
import functools

import jax
import jax.numpy as jnp
from jax import lax
from jax.experimental import pallas as pl
from jax.experimental.pallas import tpu as pltpu
from jax.experimental.pallas import tpu_sc as plsc


@functools.cache
def _make_embed(T, B, D, V):
    info = plsc.get_sparse_core_info()
    nc, ns = info.num_cores, info.num_subcores
    NW = nc * ns
    CH = B // NW
    NCB = CH // 128
    NR = D // 8
    SKEW = CH + 9
    mesh = plsc.VectorSubcoreMesh(core_axis_name="c", subcore_axis_name="s")

    @functools.partial(
        pl.kernel,
        mesh=mesh,
        out_type=jax.ShapeDtypeStruct((T, NR, B // 128, 8, 128), jnp.float32),
        scratch_types=[
            pltpu.VMEM((2, CH), jnp.int32),
            pltpu.VMEM((2, CH, D), jnp.float32),
            pltpu.VMEM((2, D, SKEW), jnp.float32),
            pltpu.SemaphoreType.DMA,
            pltpu.SemaphoreType.DMA,
            pltpu.SemaphoreType.DMA,
        ],
        compiler_params=pltpu.CompilerParams(
            use_tc_tiling_on_sc=False, needs_layout_passes=False
        ),
    )
    def k(idx_hbm, table_hbm, out_hbm, idx_v, rows_v, tr_v, isem, gsem, osem):
        wid = lax.axis_index("s") * nc + lax.axis_index("c")
        base_b = wid * CH

        def idx_src(t):
            return idx_hbm.at[pl.ds(t * B + base_b, CH)]

        pltpu.async_copy(idx_src(0), idx_v.at[0], isem)

        def per_t(t, carry):
            b = lax.rem(t, 2)
            p = 1 - b

            @pl.when(t >= 2)
            def _():
                for r in range(NR):
                    for c in range(NCB):
                        pltpu.make_async_copy(
                            tr_v.at[b, pl.ds(r * 8, 8), pl.ds(c * 128, 128)],
                            out_hbm.at[0, r, wid * NCB + c],
                            osem,
                        ).wait()

            @pl.when(t < T)
            def _():
                pltpu.make_async_copy(idx_src(t), idx_v.at[b], isem).wait()
                pltpu.async_copy(table_hbm.at[idx_v.at[b]], rows_v.at[b], gsem)

            @pl.when(t >= 1)
            def _():
                pltpu.make_async_copy(
                    table_hbm.at[idx_v.at[p]], rows_v.at[p], gsem
                ).wait()

                def tbody(i, c2):
                    for u in range(8):
                        bb = i * 8 + u
                        b_idx = jnp.full((16,), bb, jnp.int32)
                        for j in range(D // 16):
                            x = rows_v[p, bb, pl.ds(j * 16, 16)]
                            d_idx = lax.iota(jnp.int32, 16) + (j * 16)
                            plsc.store_scatter(tr_v.at[p], [d_idx, b_idx], x)
                    return c2

                lax.fori_loop(0, CH // 8, tbody, 0)

                for r in range(NR):
                    for c in range(NCB):
                        pltpu.async_copy(
                            tr_v.at[p, pl.ds(r * 8, 8), pl.ds(c * 128, 128)],
                            out_hbm.at[t - 1, r, wid * NCB + c],
                            osem,
                        )

            @pl.when(t + 1 < T)
            def _():
                pltpu.async_copy(idx_src(t + 1), idx_v.at[p], isem)

            return carry

        lax.fori_loop(0, T + 1, per_t, 0)

        for r in range(NR):
            for c in range(NCB):
                pltpu.make_async_copy(
                    tr_v.at[0, pl.ds(r * 8, 8), pl.ds(c * 128, 128)],
                    out_hbm.at[0, r, wid * NCB + c],
                    osem,
                ).wait()

    return k


def kernel(X, W):
    B, T = X.shape
    V, D = W.shape
    idx = X.transpose(1, 0).reshape(T * B).astype(jnp.int32)
    out5 = _make_embed(T, B, D, V)(idx, W)
    return out5.transpose(2, 4, 0, 1, 3).reshape(B, T, D)
